# Initial kernel scaffold; baseline (speedup 1.0000x reference)
import jax
import jax.numpy as jnp
from jax.experimental import pallas as pl


def kernel(positions, neighbors):
    raise NotImplementedError("write your pallas kernel here")


# speedup vs baseline: 180.1842x; 180.1842x over previous
import functools

import jax
import jax.numpy as jnp
from jax import lax
from jax.experimental import pallas as pl
from jax.experimental.pallas import tpu as pltpu
from jax.experimental.pallas import tpu_sc as plsc

N_BATCH = 16
N_ATOMS = 8192
NBH = 64
NW = 32
PER_W = N_BATCH * N_ATOMS * NBH // NW
CHUNK = 16384
N_CHUNKS = PER_W // CHUNK
ATOMS_PER_CHUNK = CHUNK // NBH
L = 16


def _safe_dist(sq):
    pos = sq > 0.0
    safe = jnp.where(pos, sq, 1.0)
    i = plsc.bitcast(safe, jnp.int32)
    y = plsc.bitcast(jnp.int32(0x5F3759DF) - (i >> 1), jnp.float32)
    xh = 0.5 * safe
    y = y * (1.5 - xh * y * y)
    y = y * (1.5 - xh * y * y)
    y = y * (1.5 - xh * y * y)
    return jnp.where(pos, safe * y, 0.0)


def _distances_sc(xs, ys, zs, nbr):
    mesh = plsc.VectorSubcoreMesh(core_axis_name="c", subcore_axis_name="s")

    @functools.partial(
        pl.kernel,
        out_type=jax.ShapeDtypeStruct((N_BATCH * N_ATOMS * NBH,), jnp.float32),
        mesh=mesh,
        scratch_types=[
            pltpu.VMEM((N_ATOMS,), jnp.float32),
            pltpu.VMEM((N_ATOMS,), jnp.float32),
            pltpu.VMEM((N_ATOMS,), jnp.float32),
            pltpu.VMEM((CHUNK,), jnp.int32),
            pltpu.VMEM((CHUNK,), jnp.float32),
        ],
        compiler_params=pltpu.CompilerParams(needs_layout_passes=False),
    )
    def k(xs_h, ys_h, zs_h, nbr_h, out_h, x_v, y_v, z_v, idx_v, o_v):
        cid = lax.axis_index("c")
        sid = lax.axis_index("s")
        wid = cid * 16 + sid
        b = wid // 2
        half = wid % 2
        tb = b * N_ATOMS
        pltpu.sync_copy(xs_h.at[pl.ds(tb, N_ATOMS)], x_v)
        pltpu.sync_copy(ys_h.at[pl.ds(tb, N_ATOMS)], y_v)
        pltpu.sync_copy(zs_h.at[pl.ds(tb, N_ATOMS)], z_v)
        region = wid * PER_W

        def chunk_body(c, carry):
            base = region + c * CHUNK
            pltpu.sync_copy(nbr_h.at[pl.ds(base, CHUNK)], idx_v)
            atom0 = half * (N_ATOMS // 2) + c * ATOMS_PER_CHUNK

            def atom_body(a, carry2):
                cidx = jnp.full((L,), atom0 + a, jnp.int32)
                cx = plsc.load_gather(x_v, [cidx])
                cy = plsc.load_gather(y_v, [cidx])
                cz = plsc.load_gather(z_v, [cidx])
                for j in range(NBH // L):
                    off = a * NBH + j * L
                    nb = idx_v[pl.ds(off, L)]
                    dx = plsc.load_gather(x_v, [nb]) - cx
                    dy = plsc.load_gather(y_v, [nb]) - cy
                    dz = plsc.load_gather(z_v, [nb]) - cz
                    sq = dx * dx + dy * dy + dz * dz
                    o_v[pl.ds(off, L)] = _safe_dist(sq)
                return carry2

            lax.fori_loop(0, ATOMS_PER_CHUNK, atom_body, 0)
            pltpu.sync_copy(o_v, out_h.at[pl.ds(base, CHUNK)])
            return carry

        lax.fori_loop(0, N_CHUNKS, chunk_body, 0)

    return k(xs, ys, zs, nbr)


def kernel(positions, neighbors):
    xs = positions[:, :, 0].reshape(-1)
    ys = positions[:, :, 1].reshape(-1)
    zs = positions[:, :, 2].reshape(-1)
    nbr = neighbors.astype(jnp.int32).reshape(-1)
    out = _distances_sc(xs, ys, zs, nbr)
    return out.reshape(N_BATCH, N_ATOMS, NBH)


# speedup vs baseline: 468.6915x; 2.6012x over previous
import functools

import jax
import jax.numpy as jnp
from jax import lax
from jax.experimental import pallas as pl
from jax.experimental.pallas import tpu as pltpu
from jax.experimental.pallas import tpu_sc as plsc

N_BATCH = 16
N_ATOMS = 8192
NBH = 64
NW = 32
PER_W = N_BATCH * N_ATOMS * NBH // NW
CHUNK = 16384
N_CHUNKS = PER_W // CHUNK
ATOMS_PER_CHUNK = CHUNK // NBH
L = 16


def _safe_dist(sq):
    safe = jnp.maximum(sq, 1e-30)
    i = plsc.bitcast(safe, jnp.int32)
    y = plsc.bitcast(jnp.int32(0x5F3759DF) - (i >> 1), jnp.float32)
    xh = 0.5 * safe
    y = y * (1.5 - xh * y * y)
    y = y * (1.5 - xh * y * y)
    return sq * y


def _distances_sc(xs, ys, zs, nbr):
    mesh = plsc.VectorSubcoreMesh(core_axis_name="c", subcore_axis_name="s")

    @functools.partial(
        pl.kernel,
        out_type=jax.ShapeDtypeStruct((N_BATCH * N_ATOMS * NBH,), jnp.float32),
        mesh=mesh,
        scratch_types=[
            pltpu.VMEM((N_ATOMS,), jnp.float32),
            pltpu.VMEM((N_ATOMS,), jnp.float32),
            pltpu.VMEM((N_ATOMS,), jnp.float32),
            [pltpu.VMEM((CHUNK,), jnp.int32) for _ in range(2)],
            [pltpu.VMEM((CHUNK,), jnp.float32) for _ in range(2)],
            [pltpu.SemaphoreType.DMA for _ in range(2)],
            [pltpu.SemaphoreType.DMA for _ in range(2)],
        ],
        compiler_params=pltpu.CompilerParams(needs_layout_passes=False),
    )
    def k(xs_h, ys_h, zs_h, nbr_h, out_h, x_v, y_v, z_v, idx_v, o_v,
          in_sem, out_sem):
        cid = lax.axis_index("c")
        sid = lax.axis_index("s")
        wid = cid * 16 + sid
        b = wid // 2
        half = wid % 2
        tb = b * N_ATOMS
        pltpu.sync_copy(xs_h.at[pl.ds(tb, N_ATOMS)], x_v)
        pltpu.sync_copy(ys_h.at[pl.ds(tb, N_ATOMS)], y_v)
        pltpu.sync_copy(zs_h.at[pl.ds(tb, N_ATOMS)], z_v)
        region = wid * PER_W

        for s in range(2):
            pltpu.async_copy(
                nbr_h.at[pl.ds(region + s * CHUNK, CHUNK)], idx_v[s],
                in_sem[s])

        def pair_body(g, carry):
            for s in range(2):
                c = 2 * g + s
                base = region + c * CHUNK
                idx_c = idx_v[s]
                o_c = o_v[s]
                pltpu.make_async_copy(
                    nbr_h.at[pl.ds(base, CHUNK)], idx_c, in_sem[s]).wait()

                @pl.when(c >= 2)
                def _():
                    pltpu.make_async_copy(
                        o_c, out_h.at[pl.ds(base, CHUNK)],
                        out_sem[s]).wait()

                atom0 = half * (N_ATOMS // 2) + c * ATOMS_PER_CHUNK

                @plsc.parallel_loop(0, ATOMS_PER_CHUNK, unroll=2)
                def atom_body(a):
                    cidx = jnp.full((L,), atom0 + a, jnp.int32)
                    cx = plsc.load_gather(x_v, [cidx])
                    cy = plsc.load_gather(y_v, [cidx])
                    cz = plsc.load_gather(z_v, [cidx])
                    for j in range(NBH // L):
                        off = a * NBH + j * L
                        nb = idx_c[pl.ds(off, L)]
                        dx = plsc.load_gather(x_v, [nb]) - cx
                        dy = plsc.load_gather(y_v, [nb]) - cy
                        dz = plsc.load_gather(z_v, [nb]) - cz
                        sq = dx * dx + dy * dy + dz * dz
                        o_c[pl.ds(off, L)] = _safe_dist(sq)

                pltpu.async_copy(o_c, out_h.at[pl.ds(base, CHUNK)],
                                 out_sem[s])

                @pl.when(c + 2 < N_CHUNKS)
                def _():
                    pltpu.async_copy(
                        nbr_h.at[pl.ds(base + 2 * CHUNK, CHUNK)], idx_c,
                        in_sem[s])

            return carry

        lax.fori_loop(0, N_CHUNKS // 2, pair_body, 0)
        for s in range(2):
            pltpu.make_async_copy(
                o_v[s], out_h.at[pl.ds(region, CHUNK)], out_sem[s]).wait()

    return k(xs, ys, zs, nbr)


def kernel(positions, neighbors):
    xs = positions[:, :, 0].reshape(-1)
    ys = positions[:, :, 1].reshape(-1)
    zs = positions[:, :, 2].reshape(-1)
    nbr = neighbors.astype(jnp.int32).reshape(-1)
    out = _distances_sc(xs, ys, zs, nbr)
    return out.reshape(N_BATCH, N_ATOMS, NBH)
